# baseline (speedup 1.0000x reference)
import functools

import jax
import jax.numpy as jnp
from jax import lax
from jax.experimental import pallas as pl
from jax.experimental.pallas import tpu as pltpu
from jax.experimental.pallas import tpu_sc as plsc

E = 8
TOP_K = 2
ALPHA = 0.01
D_MODEL = 1024
D_FF = 2816
N_TOK = 2048
N_ASSIGN = N_TOK * TOP_K

F_BLK = 256
N_F = D_FF // F_BLK
B_R = 256
NB = N_ASSIGN // B_R
T_ITEMS = NB + E - 1

NW = 32
CH = N_ASSIGN // NW
TOKW = N_TOK // NW
NVR = CH // 16


def _router_body(x_ref, wr_ref, idx_ref, p_ref, counts_ref, aux_ref,
                 starts_ref):
    x = x_ref[...]
    wr = wr_ref[...]
    logits = jax.lax.dot_general(
        x, wr, (((1,), (1,)), ((), ())),
        preferred_element_type=jnp.float32)
    m = jnp.max(logits, axis=-1, keepdims=True)
    ex = jnp.exp(logits - m)
    s = jnp.sum(ex, axis=-1, keepdims=True)
    probs = ex / s

    e_iota = jax.lax.broadcasted_iota(jnp.int32, (N_TOK, E), 1)
    v1 = jnp.max(probs, axis=-1, keepdims=True)
    i1 = jnp.min(jnp.where(probs == v1, e_iota, E), axis=-1, keepdims=True)
    probs_m = jnp.where(e_iota == i1, -1.0, probs)
    v2 = jnp.max(probs_m, axis=-1, keepdims=True)
    i2 = jnp.min(jnp.where(probs_m == v2, e_iota, E), axis=-1, keepdims=True)

    tsum = v1 + v2
    idx_ref[...] = jnp.concatenate([i1, i2], axis=1)
    p_ref[...] = jnp.concatenate([v1 / tsum, v2 / tsum], axis=1)

    sel1 = (e_iota == i1)
    sel2 = (e_iota == i2)
    cnt = sel1.astype(jnp.float32) + sel2.astype(jnp.float32)
    counts = jnp.sum(cnt, axis=0, keepdims=True)
    counts_ref[...] = counts.astype(jnp.int32)

    t_iota = jax.lax.broadcasted_iota(jnp.int32, (NW, N_TOK), 1)
    w_iota = jax.lax.broadcasted_iota(jnp.int32, (NW, N_TOK), 0)
    mpre = (t_iota < w_iota * TOKW).astype(jnp.float32)
    prefix = jax.lax.dot_general(
        mpre, cnt, (((1,), (0,)), ((), ())),
        precision=jax.lax.Precision.HIGHEST,
        preferred_element_type=jnp.float32)
    lt = (jax.lax.broadcasted_iota(jnp.int32, (E, E), 0) <
          jax.lax.broadcasted_iota(jnp.int32, (E, E), 1)).astype(jnp.float32)
    off = jax.lax.dot_general(
        counts, lt, (((1,), (0,)), ((), ())),
        precision=jax.lax.Precision.HIGHEST,
        preferred_element_type=jnp.float32)
    starts = (prefix + off).astype(jnp.int32)
    starts_ref[...] = jnp.concatenate(
        [starts, jnp.zeros((NW, 16 - E), jnp.int32)], axis=1)
    p_mean = jnp.mean(probs, axis=0, keepdims=True)
    f_i = counts / float(N_TOK * TOP_K)
    aux_ref[...] = (ALPHA * E) * jnp.sum(f_i * p_mean, keepdims=True).reshape(1, 1)


def _router_call(xf, w_router):
    return pl.pallas_call(
        _router_body,
        out_shape=(
            jax.ShapeDtypeStruct((N_TOK, TOP_K), jnp.int32),
            jax.ShapeDtypeStruct((N_TOK, TOP_K), jnp.float32),
            jax.ShapeDtypeStruct((1, E), jnp.int32),
            jax.ShapeDtypeStruct((1, 1), jnp.float32),
            jax.ShapeDtypeStruct((NW, 16), jnp.int32),
        ),
    )(xf, w_router)


IOTA16 = None


def _cumsum16(x, iota16):
    for rshift in (1, 2, 4, 8):
        idx = jnp.maximum(iota16 - rshift, 0)
        g = x.at[idx].get(mode="promise_in_bounds")
        x = x + jnp.where(iota16 >= rshift, g, 0)
    return x


def _splat_last(x, iota16):
    return x.at[iota16 * 0 + 15].get(mode="promise_in_bounds")


def _dispatch_body(idxf, x2, starts, xs, posj,
                   idx_v, run_v, pos_v, pe_v, po_v, rows_v, sem):
    c = lax.axis_index("c")
    s = lax.axis_index("s")
    wid = s * 2 + c
    base = wid * CH
    tok0 = wid * TOKW

    pltpu.sync_copy(idxf.at[pl.ds(base, CH)], idx_v)
    pltpu.sync_copy(starts.at[wid], run_v)
    pltpu.sync_copy(x2.at[pl.ds(tok0, TOKW)], rows_v)

    iota = lax.iota(jnp.int32, 16)
    zero = jnp.zeros((16,), jnp.int32)
    rv = run_v[...]
    run = [rv.at[iota * 0 + e].get(mode="promise_in_bounds")
           for e in range(E)]

    ps = []
    for u in range(NVR):
        ev = idx_v[pl.ds(u * 16, 16)]
        p = zero
        for e in range(E):
            m = ev == e
            mi = jnp.where(m, 1, 0)
            ci = _cumsum16(mi, iota)
            p = jnp.where(m, run[e] + ci - mi, p)
            run[e] = run[e] + _splat_last(ci, iota)
        pos_v[pl.ds(u * 16, 16)] = p
        ps.append(p)

    lo_half = iota < 8
    idx_a = jnp.minimum(2 * iota, 15)
    idx_b = jnp.clip(2 * (iota - 8), 0, 15)
    for v in range(NVR // 2):
        p0, p1 = ps[2 * v], ps[2 * v + 1]
        ea = p0.at[idx_a].get(mode="promise_in_bounds")
        eb = p1.at[idx_b].get(mode="promise_in_bounds")
        pe_v[pl.ds(v * 16, 16)] = jnp.where(lo_half, ea, eb)
        oa = p0.at[jnp.minimum(idx_a + 1, 15)].get(mode="promise_in_bounds")
        ob = p1.at[jnp.minimum(idx_b + 1, 15)].get(mode="promise_in_bounds")
        po_v[pl.ds(v * 16, 16)] = jnp.where(lo_half, oa, ob)

    pltpu.sync_copy(pos_v, posj.at[pl.ds(base, CH)])

    pltpu.sync_copy(rows_v, xs.at[pe_v])
    pltpu.sync_copy(rows_v, xs.at[po_v])


def _dispatch_call(idx_flat, x2, starts):
    mesh = plsc.VectorSubcoreMesh(core_axis_name="c", subcore_axis_name="s")
    return pl.kernel(
        _dispatch_body,
        out_type=(
            jax.ShapeDtypeStruct((N_ASSIGN, D_MODEL), jnp.float32),
            jax.ShapeDtypeStruct((N_ASSIGN,), jnp.int32),
        ),
        mesh=mesh,
        scratch_types=[
            pltpu.VMEM((CH,), jnp.int32),
            pltpu.VMEM((16,), jnp.int32),
            pltpu.VMEM((CH,), jnp.int32),
            pltpu.VMEM((TOKW,), jnp.int32),
            pltpu.VMEM((TOKW,), jnp.int32),
            pltpu.VMEM((TOKW, D_MODEL), jnp.float32),
            pltpu.SemaphoreType.DMA,
        ],
    )(idx_flat, x2, starts)


def _ffn_body(bids_s, eids_s, glo_s, ghi_s, finit_s,
              x_ref, wg_ref, wu_ref, wd_ref, y_ref,
              wgb_ref, wub_ref, wdb_ref):
    f = pl.program_id(0)
    t = pl.program_id(1)

    row0 = bids_s[t] * B_R
    lo = glo_s[t] - row0
    hi = ghi_s[t] - row0
    xb = x_ref[pl.ds(row0, B_R), :].astype(jnp.bfloat16)

    wchg = (t == 0) | (eids_s[t] != eids_s[jnp.maximum(t - 1, 0)])

    @pl.when(wchg)
    def _cast():
        wgb_ref[...] = wg_ref[0].astype(jnp.bfloat16)
        wub_ref[...] = wu_ref[0].astype(jnp.bfloat16)
        wdb_ref[...] = wd_ref[0].astype(jnp.bfloat16)

    g = jnp.dot(xb, wgb_ref[...], preferred_element_type=jnp.float32)
    u = jnp.dot(xb, wub_ref[...], preferred_element_type=jnp.float32)
    h = g * jax.nn.sigmoid(g) * u
    ri = jax.lax.broadcasted_iota(jnp.int32, (B_R, F_BLK), 0)
    h = jnp.where((ri >= lo) & (ri < hi), h, 0.0)
    d = jnp.dot(h.astype(jnp.bfloat16), wdb_ref[...],
                preferred_element_type=jnp.float32)
    fresh = (f == 0) & (finit_s[t] == 1)
    prev = jnp.where(fresh, 0.0, y_ref[pl.ds(row0, B_R), :])
    y_ref[pl.ds(row0, B_R), :] = prev + d


def _ffn_call(meta, xs2, w_gate, w_up, w_down):
    bids, eids, glo, ghi, finit = meta
    grid_spec = pltpu.PrefetchScalarGridSpec(
        num_scalar_prefetch=5,
        grid=(N_F, T_ITEMS),
        in_specs=[
            pl.BlockSpec((N_ASSIGN, D_MODEL),
                         lambda f, t, bs, es, ls, hs, fs: (0, 0)),
            pl.BlockSpec((1, D_MODEL, F_BLK),
                         lambda f, t, bs, es, ls, hs, fs: (es[t], 0, f)),
            pl.BlockSpec((1, D_MODEL, F_BLK),
                         lambda f, t, bs, es, ls, hs, fs: (es[t], 0, f)),
            pl.BlockSpec((1, F_BLK, D_MODEL),
                         lambda f, t, bs, es, ls, hs, fs: (es[t], f, 0)),
        ],
        out_specs=pl.BlockSpec((N_ASSIGN, D_MODEL),
                               lambda f, t, bs, es, ls, hs, fs: (0, 0)),
        scratch_shapes=[
            pltpu.VMEM((D_MODEL, F_BLK), jnp.bfloat16),
            pltpu.VMEM((D_MODEL, F_BLK), jnp.bfloat16),
            pltpu.VMEM((F_BLK, D_MODEL), jnp.bfloat16),
        ],
    )
    return pl.pallas_call(
        _ffn_body,
        grid_spec=grid_spec,
        out_shape=jax.ShapeDtypeStruct((N_ASSIGN, D_MODEL), jnp.float32),
        compiler_params=pltpu.CompilerParams(
            dimension_semantics=("arbitrary", "arbitrary")),
    )(bids, eids, glo, ghi, finit, xs2, w_gate, w_up, w_down)


def _metadata(counts):
    off = jnp.concatenate(
        [jnp.zeros((1,), jnp.int32), jnp.cumsum(counts, dtype=jnp.int32)])
    bb = jnp.repeat(jnp.arange(NB, dtype=jnp.int32), E)
    ee = jnp.tile(jnp.arange(E, dtype=jnp.int32), NB)
    lo = jnp.maximum(off[ee], bb * B_R)
    hi = jnp.minimum(off[ee + 1], (bb + 1) * B_R)
    valid = lo < hi
    nvalid = jnp.sum(valid.astype(jnp.int32))
    rank = jnp.cumsum(valid.astype(jnp.int32)) - 1
    dest = jnp.where(valid, rank, T_ITEMS)
    bids = jnp.zeros((T_ITEMS,), jnp.int32).at[dest].set(bb, mode="drop")
    eids = jnp.zeros((T_ITEMS,), jnp.int32).at[dest].set(ee, mode="drop")
    glo = jnp.zeros((T_ITEMS,), jnp.int32).at[dest].set(lo, mode="drop")
    ghi = jnp.zeros((T_ITEMS,), jnp.int32).at[dest].set(hi, mode="drop")
    tpos = jnp.arange(T_ITEMS, dtype=jnp.int32)
    pad = tpos >= nvalid
    lb = jnp.take(bids, nvalid - 1)
    le = jnp.take(eids, nvalid - 1)
    lh = jnp.take(ghi, nvalid - 1)
    bids = jnp.where(pad, lb, bids)
    eids = jnp.where(pad, le, eids)
    glo = jnp.where(pad, lh, glo)
    ghi = jnp.where(pad, lh, ghi)
    finit = (jnp.concatenate([jnp.full((1,), -1, jnp.int32), bids[:-1]])
             != bids).astype(jnp.int32)
    return bids, eids, glo, ghi, finit


def _combine_body(y, posj, pw, out, pos_v, pw_v, rows_v, out_v, sem):
    c = lax.axis_index("c")
    s = lax.axis_index("s")
    wid = s * 2 + c
    base = wid * CH
    tok0 = wid * TOKW

    pltpu.sync_copy(pw.at[pl.ds(base, CH)], pw_v)
    iota = lax.iota(jnp.int32, 16)

    for half in range(2):
        pltpu.sync_copy(posj.at[pl.ds(base + half * 64, 64)], pos_v.at[half])
        pltpu.async_copy(y.at[pos_v.at[half]], rows_v, sem).wait()

        def token_body(i, _):
            a0 = half * 64 + 2 * i
            q = pw_v[pl.ds((a0 // 16) * 16, 16)]
            l0 = a0 % 16
            w0 = q.at[iota * 0 + l0].get(mode="promise_in_bounds")
            w1 = q.at[iota * 0 + l0 + 1].get(mode="promise_in_bounds")
            r0 = 2 * i
            for v in range(D_MODEL // 16):
                x0 = rows_v[r0, pl.ds(v * 16, 16)]
                x1 = rows_v[r0 + 1, pl.ds(v * 16, 16)]
                out_v[i, pl.ds(v * 16, 16)] = w0 * x0 + w1 * x1
            return 0

        lax.fori_loop(0, 32, token_body, 0)
        pltpu.sync_copy(out_v, out.at[pl.ds(tok0 + half * 32, 32)])


def _combine_call(y, posj, pwflat):
    mesh = plsc.VectorSubcoreMesh(core_axis_name="c", subcore_axis_name="s")
    return pl.kernel(
        _combine_body,
        out_type=jax.ShapeDtypeStruct((N_TOK, D_MODEL), jnp.float32),
        mesh=mesh,
        scratch_types=[
            pltpu.VMEM((2, 64), jnp.int32),
            pltpu.VMEM((CH,), jnp.float32),
            pltpu.VMEM((64, D_MODEL), jnp.float32),
            pltpu.VMEM((32, D_MODEL), jnp.float32),
            pltpu.SemaphoreType.DMA,
        ],
    )(y, posj, pwflat)


def kernel(x, W_router, W_gate, W_up, W_down):
    B, S, D = x.shape
    xf = x.reshape(S, D)
    idx2, p2, counts, aux, starts = _router_call(xf, W_router)

    xs, posj = _dispatch_call(idx2.reshape(N_ASSIGN), xf, starts)

    meta = _metadata(counts.reshape(E))
    y = _ffn_call(meta, xs, W_gate, W_up, W_down)

    out = _combine_call(y, posj, p2.reshape(N_ASSIGN))
    return out.reshape(B, S, D), aux.reshape(())


# speedup vs baseline: 1.2825x; 1.2825x over previous
import functools

import jax
import jax.numpy as jnp
from jax import lax
from jax.experimental import pallas as pl
from jax.experimental.pallas import tpu as pltpu
from jax.experimental.pallas import tpu_sc as plsc

E = 8
TOP_K = 2
ALPHA = 0.01
D_MODEL = 1024
D_FF = 2816
N_TOK = 2048
N_ASSIGN = N_TOK * TOP_K

F_BLK = 256
N_F = D_FF // F_BLK
B_R = 256
NB = N_ASSIGN // B_R
T_ITEMS = NB + E - 1

NW = 32
CH = N_ASSIGN // NW
TOKW = N_TOK // NW
NVR = CH // 16


def _router_body(x_ref, wr_ref, idx_ref, p_ref, counts_ref, aux_ref,
                 starts_ref):
    x = x_ref[...]
    wr = wr_ref[...]
    logits = jax.lax.dot_general(
        x, wr, (((1,), (1,)), ((), ())),
        preferred_element_type=jnp.float32)
    m = jnp.max(logits, axis=-1, keepdims=True)
    ex = jnp.exp(logits - m)
    s = jnp.sum(ex, axis=-1, keepdims=True)
    probs = ex / s

    e_iota = jax.lax.broadcasted_iota(jnp.int32, (N_TOK, E), 1)
    v1 = jnp.max(probs, axis=-1, keepdims=True)
    i1 = jnp.min(jnp.where(probs == v1, e_iota, E), axis=-1, keepdims=True)
    probs_m = jnp.where(e_iota == i1, -1.0, probs)
    v2 = jnp.max(probs_m, axis=-1, keepdims=True)
    i2 = jnp.min(jnp.where(probs_m == v2, e_iota, E), axis=-1, keepdims=True)

    tsum = v1 + v2
    idx_ref[...] = jnp.concatenate([i1, i2], axis=1)
    p_ref[...] = jnp.concatenate([v1 / tsum, v2 / tsum], axis=1)

    sel1 = (e_iota == i1)
    sel2 = (e_iota == i2)
    cnt = sel1.astype(jnp.float32) + sel2.astype(jnp.float32)
    counts = jnp.sum(cnt, axis=0, keepdims=True)
    counts_ref[...] = counts.astype(jnp.int32)

    t_iota = jax.lax.broadcasted_iota(jnp.int32, (NW, N_TOK), 1)
    w_iota = jax.lax.broadcasted_iota(jnp.int32, (NW, N_TOK), 0)
    mpre = (t_iota < w_iota * TOKW).astype(jnp.float32)
    prefix = jax.lax.dot_general(
        mpre, cnt, (((1,), (0,)), ((), ())),
        precision=jax.lax.Precision.HIGHEST,
        preferred_element_type=jnp.float32)
    lt = (jax.lax.broadcasted_iota(jnp.int32, (E, E), 0) <
          jax.lax.broadcasted_iota(jnp.int32, (E, E), 1)).astype(jnp.float32)
    off = jax.lax.dot_general(
        counts, lt, (((1,), (0,)), ((), ())),
        precision=jax.lax.Precision.HIGHEST,
        preferred_element_type=jnp.float32)
    starts = (prefix + off).astype(jnp.int32)
    starts_ref[...] = jnp.concatenate(
        [starts, jnp.zeros((NW, 16 - E), jnp.int32)], axis=1)
    p_mean = jnp.mean(probs, axis=0, keepdims=True)
    f_i = counts / float(N_TOK * TOP_K)
    aux_ref[...] = (ALPHA * E) * jnp.sum(f_i * p_mean, keepdims=True).reshape(1, 1)


def _router_call(xf, w_router):
    return pl.pallas_call(
        _router_body,
        out_shape=(
            jax.ShapeDtypeStruct((N_TOK, TOP_K), jnp.int32),
            jax.ShapeDtypeStruct((N_TOK, TOP_K), jnp.float32),
            jax.ShapeDtypeStruct((1, E), jnp.int32),
            jax.ShapeDtypeStruct((1, 1), jnp.float32),
            jax.ShapeDtypeStruct((NW, 16), jnp.int32),
        ),
    )(xf, w_router)


IOTA16 = None


def _cumsum16(x, iota16):
    for rshift in (1, 2, 4, 8):
        idx = jnp.maximum(iota16 - rshift, 0)
        g = x.at[idx].get(mode="promise_in_bounds")
        x = x + jnp.where(iota16 >= rshift, g, 0)
    return x


def _splat_last(x, iota16):
    return x.at[iota16 * 0 + 15].get(mode="promise_in_bounds")


def _dispatch_body(idxf, x2, starts, xs, posj,
                   idx_v, run_v, pos_v, pe_v, po_v, rows_v, sem):
    c = lax.axis_index("c")
    s = lax.axis_index("s")
    wid = s * 2 + c
    base = wid * CH
    tok0 = wid * TOKW

    pltpu.sync_copy(idxf.at[pl.ds(base, CH)], idx_v)
    pltpu.sync_copy(starts.at[wid], run_v)
    pltpu.sync_copy(x2.at[pl.ds(tok0, TOKW)], rows_v)

    iota = lax.iota(jnp.int32, 16)
    zero = jnp.zeros((16,), jnp.int32)
    rv = run_v[...]
    run = [rv.at[iota * 0 + e].get(mode="promise_in_bounds")
           for e in range(E)]

    ps = []
    for u in range(NVR):
        ev = idx_v[pl.ds(u * 16, 16)]
        p = zero
        for e in range(E):
            m = ev == e
            mi = jnp.where(m, 1, 0)
            ci = _cumsum16(mi, iota)
            p = jnp.where(m, run[e] + ci - mi, p)
            run[e] = run[e] + _splat_last(ci, iota)
        pos_v[pl.ds(u * 16, 16)] = p
        ps.append(p)

    lo_half = iota < 8
    idx_a = jnp.minimum(2 * iota, 15)
    idx_b = jnp.clip(2 * (iota - 8), 0, 15)
    for v in range(NVR // 2):
        p0, p1 = ps[2 * v], ps[2 * v + 1]
        ea = p0.at[idx_a].get(mode="promise_in_bounds")
        eb = p1.at[idx_b].get(mode="promise_in_bounds")
        pe_v[pl.ds(v * 16, 16)] = jnp.where(lo_half, ea, eb)
        oa = p0.at[jnp.minimum(idx_a + 1, 15)].get(mode="promise_in_bounds")
        ob = p1.at[jnp.minimum(idx_b + 1, 15)].get(mode="promise_in_bounds")
        po_v[pl.ds(v * 16, 16)] = jnp.where(lo_half, oa, ob)

    pltpu.sync_copy(pos_v, posj.at[pl.ds(base, CH)])

    pltpu.sync_copy(rows_v, xs.at[pe_v])
    pltpu.sync_copy(rows_v, xs.at[po_v])


def _dispatch_call(idx_flat, x2, starts):
    mesh = plsc.VectorSubcoreMesh(core_axis_name="c", subcore_axis_name="s")
    return pl.kernel(
        _dispatch_body,
        out_type=(
            jax.ShapeDtypeStruct((N_ASSIGN, D_MODEL), jnp.float32),
            jax.ShapeDtypeStruct((N_ASSIGN,), jnp.int32),
        ),
        mesh=mesh,
        scratch_types=[
            pltpu.VMEM((CH,), jnp.int32),
            pltpu.VMEM((16,), jnp.int32),
            pltpu.VMEM((CH,), jnp.int32),
            pltpu.VMEM((TOKW,), jnp.int32),
            pltpu.VMEM((TOKW,), jnp.int32),
            pltpu.VMEM((TOKW, D_MODEL), jnp.float32),
            pltpu.SemaphoreType.DMA,
        ],
    )(idx_flat, x2, starts)


FH = D_FF // 2


def _ffn_a_body(bids_s, eids_s, glo_s, ghi_s, finit_s,
                x_ref, wg_ref, wu_ref, h_ref, wgb_ref, wub_ref):
    t = pl.program_id(1)
    row0 = bids_s[t] * B_R
    lo = glo_s[t] - row0
    hi = ghi_s[t] - row0
    xb = x_ref[...].astype(jnp.bfloat16)

    wchg = (t == 0) | (eids_s[t] != eids_s[jnp.maximum(t - 1, 0)])

    @pl.when(wchg)
    def _cast():
        wgb_ref[...] = wg_ref[0].astype(jnp.bfloat16)
        wub_ref[...] = wu_ref[0].astype(jnp.bfloat16)

    g = jnp.dot(xb, wgb_ref[...], preferred_element_type=jnp.float32)
    u = jnp.dot(xb, wub_ref[...], preferred_element_type=jnp.float32)
    h = g * jax.nn.sigmoid(g) * u
    ri = jax.lax.broadcasted_iota(jnp.int32, (B_R, FH), 0)
    h = jnp.where((ri >= lo) & (ri < hi), h, 0.0).astype(jnp.bfloat16)
    fresh = finit_s[t] == 1
    prev = jnp.where(fresh, jnp.bfloat16(0), h_ref[...])
    h_ref[...] = prev + h


def _ffn_b_body(bids_s, eids_s, glo_s, ghi_s, finit_s,
                h_ref, wd_ref, y_ref, wdb_ref):
    fh = pl.program_id(0)
    t = pl.program_id(1)
    row0 = bids_s[t] * B_R
    lo = glo_s[t] - row0
    hi = ghi_s[t] - row0
    wchg = (t == 0) | (eids_s[t] != eids_s[jnp.maximum(t - 1, 0)])

    @pl.when(wchg)
    def _cast():
        wdb_ref[...] = wd_ref[0].astype(jnp.bfloat16)

    ri = jax.lax.broadcasted_iota(jnp.int32, (B_R, FH), 0)
    hm = jnp.where((ri >= lo) & (ri < hi), h_ref[...], jnp.bfloat16(0))
    d = jnp.dot(hm, wdb_ref[...], preferred_element_type=jnp.float32)
    fresh = (fh == 0) & (finit_s[t] == 1)
    prev = jnp.where(fresh, 0.0, y_ref[pl.ds(row0, B_R), :])
    y_ref[pl.ds(row0, B_R), :] = prev + d


def _ffn_call(meta, xs2, w_gate, w_up, w_down):
    bids, eids, glo, ghi, finit = meta
    grid_a = pltpu.PrefetchScalarGridSpec(
        num_scalar_prefetch=5,
        grid=(2, T_ITEMS),
        in_specs=[
            pl.BlockSpec((B_R, D_MODEL),
                         lambda f, t, bs, es, ls, hs, fs: (bs[t], 0)),
            pl.BlockSpec((1, D_MODEL, FH),
                         lambda f, t, bs, es, ls, hs, fs: (es[t], 0, f)),
            pl.BlockSpec((1, D_MODEL, FH),
                         lambda f, t, bs, es, ls, hs, fs: (es[t], 0, f)),
        ],
        out_specs=pl.BlockSpec((B_R, FH),
                               lambda f, t, bs, es, ls, hs, fs: (bs[t], f)),
        scratch_shapes=[
            pltpu.VMEM((D_MODEL, FH), jnp.bfloat16),
            pltpu.VMEM((D_MODEL, FH), jnp.bfloat16),
        ],
    )
    h_full = pl.pallas_call(
        _ffn_a_body,
        grid_spec=grid_a,
        out_shape=jax.ShapeDtypeStruct((N_ASSIGN, D_FF), jnp.bfloat16),
        compiler_params=pltpu.CompilerParams(
            dimension_semantics=("arbitrary", "arbitrary")),
    )(bids, eids, glo, ghi, finit, xs2, w_gate, w_up)

    grid_b = pltpu.PrefetchScalarGridSpec(
        num_scalar_prefetch=5,
        grid=(2, T_ITEMS),
        in_specs=[
            pl.BlockSpec((B_R, FH),
                         lambda f, t, bs, es, ls, hs, fs: (bs[t], f)),
            pl.BlockSpec((1, FH, D_MODEL),
                         lambda f, t, bs, es, ls, hs, fs: (es[t], f, 0)),
        ],
        out_specs=pl.BlockSpec((N_ASSIGN, D_MODEL),
                               lambda f, t, bs, es, ls, hs, fs: (0, 0)),
        scratch_shapes=[
            pltpu.VMEM((FH, D_MODEL), jnp.bfloat16),
        ],
    )
    return pl.pallas_call(
        _ffn_b_body,
        grid_spec=grid_b,
        out_shape=jax.ShapeDtypeStruct((N_ASSIGN, D_MODEL), jnp.float32),
        compiler_params=pltpu.CompilerParams(
            dimension_semantics=("arbitrary", "arbitrary")),
    )(bids, eids, glo, ghi, finit, h_full, w_down)


def _metadata(counts):
    off = jnp.concatenate(
        [jnp.zeros((1,), jnp.int32), jnp.cumsum(counts, dtype=jnp.int32)])
    bb = jnp.repeat(jnp.arange(NB, dtype=jnp.int32), E)
    ee = jnp.tile(jnp.arange(E, dtype=jnp.int32), NB)
    lo = jnp.maximum(off[ee], bb * B_R)
    hi = jnp.minimum(off[ee + 1], (bb + 1) * B_R)
    valid = lo < hi
    nvalid = jnp.sum(valid.astype(jnp.int32))
    rank = jnp.cumsum(valid.astype(jnp.int32)) - 1
    dest = jnp.where(valid, rank, T_ITEMS)
    bids = jnp.zeros((T_ITEMS,), jnp.int32).at[dest].set(bb, mode="drop")
    eids = jnp.zeros((T_ITEMS,), jnp.int32).at[dest].set(ee, mode="drop")
    glo = jnp.zeros((T_ITEMS,), jnp.int32).at[dest].set(lo, mode="drop")
    ghi = jnp.zeros((T_ITEMS,), jnp.int32).at[dest].set(hi, mode="drop")
    tpos = jnp.arange(T_ITEMS, dtype=jnp.int32)
    pad = tpos >= nvalid
    lb = jnp.take(bids, nvalid - 1)
    le = jnp.take(eids, nvalid - 1)
    lh = jnp.take(ghi, nvalid - 1)
    bids = jnp.where(pad, lb, bids)
    eids = jnp.where(pad, le, eids)
    glo = jnp.where(pad, lh, glo)
    ghi = jnp.where(pad, lh, ghi)
    finit = (jnp.concatenate([jnp.full((1,), -1, jnp.int32), bids[:-1]])
             != bids).astype(jnp.int32)
    return bids, eids, glo, ghi, finit


def _combine_body(y, posj, pw, out, pos_v, pw_v, rows_v, out_v, sem):
    c = lax.axis_index("c")
    s = lax.axis_index("s")
    wid = s * 2 + c
    base = wid * CH
    tok0 = wid * TOKW

    pltpu.sync_copy(pw.at[pl.ds(base, CH)], pw_v)
    iota = lax.iota(jnp.int32, 16)

    for half in range(2):
        pltpu.sync_copy(posj.at[pl.ds(base + half * 64, 64)], pos_v.at[half])
        pltpu.async_copy(y.at[pos_v.at[half]], rows_v, sem).wait()

        def token_body(i, _):
            a0 = half * 64 + 2 * i
            q = pw_v[pl.ds((a0 // 16) * 16, 16)]
            l0 = a0 % 16
            w0 = q.at[iota * 0 + l0].get(mode="promise_in_bounds")
            w1 = q.at[iota * 0 + l0 + 1].get(mode="promise_in_bounds")
            r0 = 2 * i
            for v in range(D_MODEL // 16):
                x0 = rows_v[r0, pl.ds(v * 16, 16)]
                x1 = rows_v[r0 + 1, pl.ds(v * 16, 16)]
                out_v[i, pl.ds(v * 16, 16)] = w0 * x0 + w1 * x1
            return 0

        lax.fori_loop(0, 32, token_body, 0)
        pltpu.sync_copy(out_v, out.at[pl.ds(tok0 + half * 32, 32)])


def _combine_call(y, posj, pwflat):
    mesh = plsc.VectorSubcoreMesh(core_axis_name="c", subcore_axis_name="s")
    return pl.kernel(
        _combine_body,
        out_type=jax.ShapeDtypeStruct((N_TOK, D_MODEL), jnp.float32),
        mesh=mesh,
        scratch_types=[
            pltpu.VMEM((2, 64), jnp.int32),
            pltpu.VMEM((CH,), jnp.float32),
            pltpu.VMEM((64, D_MODEL), jnp.float32),
            pltpu.VMEM((32, D_MODEL), jnp.float32),
            pltpu.SemaphoreType.DMA,
        ],
    )(y, posj, pwflat)


def kernel(x, W_router, W_gate, W_up, W_down):
    B, S, D = x.shape
    xf = x.reshape(S, D)
    idx2, p2, counts, aux, starts = _router_call(xf, W_router)

    xs, posj = _dispatch_call(idx2.reshape(N_ASSIGN), xf, starts)

    meta = _metadata(counts.reshape(E))
    y = _ffn_call(meta, xs, W_gate, W_up, W_down)

    out = _combine_call(y, posj, p2.reshape(N_ASSIGN))
    return out.reshape(B, S, D), aux.reshape(())


# speedup vs baseline: 1.3449x; 1.0486x over previous
import functools

import jax
import jax.numpy as jnp
from jax import lax
from jax.experimental import pallas as pl
from jax.experimental.pallas import tpu as pltpu
from jax.experimental.pallas import tpu_sc as plsc

E = 8
TOP_K = 2
ALPHA = 0.01
D_MODEL = 1024
D_FF = 2816
N_TOK = 2048
N_ASSIGN = N_TOK * TOP_K

F_BLK = 256
N_F = D_FF // F_BLK
B_R = 512
NB = N_ASSIGN // B_R
T_ITEMS = NB + E - 1

NW = 32
CH = N_ASSIGN // NW
TOKW = N_TOK // NW
NVR = CH // 16


def _router_body(x_ref, wr_ref, idx_ref, p_ref, counts_ref, aux_ref,
                 starts_ref):
    x = x_ref[...]
    wr = wr_ref[...]
    logits = jax.lax.dot_general(
        x, wr, (((1,), (1,)), ((), ())),
        preferred_element_type=jnp.float32)
    m = jnp.max(logits, axis=-1, keepdims=True)
    ex = jnp.exp(logits - m)
    s = jnp.sum(ex, axis=-1, keepdims=True)
    probs = ex / s

    e_iota = jax.lax.broadcasted_iota(jnp.int32, (N_TOK, E), 1)
    v1 = jnp.max(probs, axis=-1, keepdims=True)
    i1 = jnp.min(jnp.where(probs == v1, e_iota, E), axis=-1, keepdims=True)
    probs_m = jnp.where(e_iota == i1, -1.0, probs)
    v2 = jnp.max(probs_m, axis=-1, keepdims=True)
    i2 = jnp.min(jnp.where(probs_m == v2, e_iota, E), axis=-1, keepdims=True)

    tsum = v1 + v2
    idx_ref[...] = jnp.concatenate([i1, i2], axis=1)
    p_ref[...] = jnp.concatenate([v1 / tsum, v2 / tsum], axis=1)

    sel1 = (e_iota == i1)
    sel2 = (e_iota == i2)
    cnt = sel1.astype(jnp.float32) + sel2.astype(jnp.float32)
    counts = jnp.sum(cnt, axis=0, keepdims=True)
    counts_ref[...] = counts.astype(jnp.int32)

    t_iota = jax.lax.broadcasted_iota(jnp.int32, (NW, N_TOK), 1)
    w_iota = jax.lax.broadcasted_iota(jnp.int32, (NW, N_TOK), 0)
    mpre = (t_iota < w_iota * TOKW).astype(jnp.float32)
    prefix = jax.lax.dot_general(
        mpre, cnt, (((1,), (0,)), ((), ())),
        precision=jax.lax.Precision.HIGHEST,
        preferred_element_type=jnp.float32)
    lt = (jax.lax.broadcasted_iota(jnp.int32, (E, E), 0) <
          jax.lax.broadcasted_iota(jnp.int32, (E, E), 1)).astype(jnp.float32)
    off = jax.lax.dot_general(
        counts, lt, (((1,), (0,)), ((), ())),
        precision=jax.lax.Precision.HIGHEST,
        preferred_element_type=jnp.float32)
    starts = (prefix + off).astype(jnp.int32)
    starts_ref[...] = jnp.concatenate(
        [starts, jnp.zeros((NW, 16 - E), jnp.int32)], axis=1)
    p_mean = jnp.mean(probs, axis=0, keepdims=True)
    f_i = counts / float(N_TOK * TOP_K)
    aux_ref[...] = (ALPHA * E) * jnp.sum(f_i * p_mean, keepdims=True).reshape(1, 1)


def _router_call(xf, w_router):
    return pl.pallas_call(
        _router_body,
        out_shape=(
            jax.ShapeDtypeStruct((N_TOK, TOP_K), jnp.int32),
            jax.ShapeDtypeStruct((N_TOK, TOP_K), jnp.float32),
            jax.ShapeDtypeStruct((1, E), jnp.int32),
            jax.ShapeDtypeStruct((1, 1), jnp.float32),
            jax.ShapeDtypeStruct((NW, 16), jnp.int32),
        ),
    )(xf, w_router)


IOTA16 = None


def _cumsum16(x, iota16):
    for rshift in (1, 2, 4, 8):
        idx = jnp.maximum(iota16 - rshift, 0)
        g = x.at[idx].get(mode="promise_in_bounds")
        x = x + jnp.where(iota16 >= rshift, g, 0)
    return x


def _splat_last(x, iota16):
    return x.at[iota16 * 0 + 15].get(mode="promise_in_bounds")


def _dispatch_body(idxf, x2, starts, xs, posj,
                   idx_v, run_v, pos_v, pe_v, po_v, rows_v, sem):
    c = lax.axis_index("c")
    s = lax.axis_index("s")
    wid = s * 2 + c
    base = wid * CH
    tok0 = wid * TOKW

    pltpu.sync_copy(idxf.at[pl.ds(base, CH)], idx_v)
    pltpu.sync_copy(starts.at[wid], run_v)
    pltpu.sync_copy(x2.at[pl.ds(tok0, TOKW)], rows_v)

    iota = lax.iota(jnp.int32, 16)
    zero = jnp.zeros((16,), jnp.int32)
    rv = run_v[...]
    run = [rv.at[iota * 0 + e].get(mode="promise_in_bounds")
           for e in range(E)]

    ps = []
    for u in range(NVR):
        ev = idx_v[pl.ds(u * 16, 16)]
        p = zero
        for e in range(E):
            m = ev == e
            mi = jnp.where(m, 1, 0)
            ci = _cumsum16(mi, iota)
            p = jnp.where(m, run[e] + ci - mi, p)
            run[e] = run[e] + _splat_last(ci, iota)
        pos_v[pl.ds(u * 16, 16)] = p
        ps.append(p)

    lo_half = iota < 8
    idx_a = jnp.minimum(2 * iota, 15)
    idx_b = jnp.clip(2 * (iota - 8), 0, 15)
    for v in range(NVR // 2):
        p0, p1 = ps[2 * v], ps[2 * v + 1]
        ea = p0.at[idx_a].get(mode="promise_in_bounds")
        eb = p1.at[idx_b].get(mode="promise_in_bounds")
        pe_v[pl.ds(v * 16, 16)] = jnp.where(lo_half, ea, eb)
        oa = p0.at[jnp.minimum(idx_a + 1, 15)].get(mode="promise_in_bounds")
        ob = p1.at[jnp.minimum(idx_b + 1, 15)].get(mode="promise_in_bounds")
        po_v[pl.ds(v * 16, 16)] = jnp.where(lo_half, oa, ob)

    pltpu.sync_copy(pos_v, posj.at[pl.ds(base, CH)])

    pltpu.sync_copy(rows_v, xs.at[pe_v])
    pltpu.sync_copy(rows_v, xs.at[po_v])


def _dispatch_call(idx_flat, x2, starts):
    mesh = plsc.VectorSubcoreMesh(core_axis_name="c", subcore_axis_name="s")
    return pl.kernel(
        _dispatch_body,
        out_type=(
            jax.ShapeDtypeStruct((N_ASSIGN, D_MODEL), jnp.float32),
            jax.ShapeDtypeStruct((N_ASSIGN,), jnp.int32),
        ),
        mesh=mesh,
        scratch_types=[
            pltpu.VMEM((CH,), jnp.int32),
            pltpu.VMEM((16,), jnp.int32),
            pltpu.VMEM((CH,), jnp.int32),
            pltpu.VMEM((TOKW,), jnp.int32),
            pltpu.VMEM((TOKW,), jnp.int32),
            pltpu.VMEM((TOKW, D_MODEL), jnp.float32),
            pltpu.SemaphoreType.DMA,
        ],
    )(idx_flat, x2, starts)


FH = D_FF // 2


def _ffn_a_body(bids_s, eids_s, glo_s, ghi_s, finit_s,
                x_ref, wg_ref, wu_ref, h_ref, wgb_ref, wub_ref):
    t = pl.program_id(1)
    row0 = bids_s[t] * B_R
    lo = glo_s[t] - row0
    hi = ghi_s[t] - row0
    xb = x_ref[...].astype(jnp.bfloat16)

    wchg = (t == 0) | (eids_s[t] != eids_s[jnp.maximum(t - 1, 0)])

    @pl.when(wchg)
    def _cast():
        wgb_ref[...] = wg_ref[0].astype(jnp.bfloat16)
        wub_ref[...] = wu_ref[0].astype(jnp.bfloat16)

    g = jnp.dot(xb, wgb_ref[...], preferred_element_type=jnp.float32)
    u = jnp.dot(xb, wub_ref[...], preferred_element_type=jnp.float32)
    h = g * jax.nn.sigmoid(g) * u
    ri = jax.lax.broadcasted_iota(jnp.int32, (B_R, FH), 0)
    h = jnp.where((ri >= lo) & (ri < hi), h, 0.0).astype(jnp.bfloat16)
    fresh = finit_s[t] == 1
    prev = jnp.where(fresh, jnp.bfloat16(0), h_ref[...])
    h_ref[...] = prev + h


def _ffn_b_body(bids_s, eids_s, glo_s, ghi_s, finit_s,
                h_ref, wd_ref, y_ref, wdb_ref):
    fh = pl.program_id(0)
    t = pl.program_id(1)
    row0 = bids_s[t] * B_R
    lo = glo_s[t] - row0
    hi = ghi_s[t] - row0
    wchg = (t == 0) | (eids_s[t] != eids_s[jnp.maximum(t - 1, 0)])

    @pl.when(wchg)
    def _cast():
        wdb_ref[...] = wd_ref[0].astype(jnp.bfloat16)

    ri = jax.lax.broadcasted_iota(jnp.int32, (B_R, FH), 0)
    hm = jnp.where((ri >= lo) & (ri < hi), h_ref[...], jnp.bfloat16(0))
    d = jnp.dot(hm, wdb_ref[...], preferred_element_type=jnp.float32)
    fresh = (fh == 0) & (finit_s[t] == 1)
    prev = jnp.where(fresh, 0.0, y_ref[pl.ds(row0, B_R), :])
    y_ref[pl.ds(row0, B_R), :] = prev + d


def _ffn_call(meta, xs2, w_gate, w_up, w_down):
    bids, eids, glo, ghi, finit = meta
    grid_a = pltpu.PrefetchScalarGridSpec(
        num_scalar_prefetch=5,
        grid=(2, T_ITEMS),
        in_specs=[
            pl.BlockSpec((B_R, D_MODEL),
                         lambda f, t, bs, es, ls, hs, fs: (bs[t], 0)),
            pl.BlockSpec((1, D_MODEL, FH),
                         lambda f, t, bs, es, ls, hs, fs: (es[t], 0, f)),
            pl.BlockSpec((1, D_MODEL, FH),
                         lambda f, t, bs, es, ls, hs, fs: (es[t], 0, f)),
        ],
        out_specs=pl.BlockSpec((B_R, FH),
                               lambda f, t, bs, es, ls, hs, fs: (bs[t], f)),
        scratch_shapes=[
            pltpu.VMEM((D_MODEL, FH), jnp.bfloat16),
            pltpu.VMEM((D_MODEL, FH), jnp.bfloat16),
        ],
    )
    h_full = pl.pallas_call(
        _ffn_a_body,
        grid_spec=grid_a,
        out_shape=jax.ShapeDtypeStruct((N_ASSIGN, D_FF), jnp.bfloat16),
        compiler_params=pltpu.CompilerParams(
            dimension_semantics=("arbitrary", "arbitrary")),
    )(bids, eids, glo, ghi, finit, xs2, w_gate, w_up)

    grid_b = pltpu.PrefetchScalarGridSpec(
        num_scalar_prefetch=5,
        grid=(2, T_ITEMS),
        in_specs=[
            pl.BlockSpec((B_R, FH),
                         lambda f, t, bs, es, ls, hs, fs: (bs[t], f)),
            pl.BlockSpec((1, FH, D_MODEL),
                         lambda f, t, bs, es, ls, hs, fs: (es[t], f, 0)),
        ],
        out_specs=pl.BlockSpec((N_ASSIGN, D_MODEL),
                               lambda f, t, bs, es, ls, hs, fs: (0, 0)),
        scratch_shapes=[
            pltpu.VMEM((FH, D_MODEL), jnp.bfloat16),
        ],
    )
    return pl.pallas_call(
        _ffn_b_body,
        grid_spec=grid_b,
        out_shape=jax.ShapeDtypeStruct((N_ASSIGN, D_MODEL), jnp.float32),
        compiler_params=pltpu.CompilerParams(
            dimension_semantics=("arbitrary", "arbitrary")),
    )(bids, eids, glo, ghi, finit, h_full, w_down)


def _metadata(counts):
    off = jnp.concatenate(
        [jnp.zeros((1,), jnp.int32), jnp.cumsum(counts, dtype=jnp.int32)])
    bb = jnp.repeat(jnp.arange(NB, dtype=jnp.int32), E)
    ee = jnp.tile(jnp.arange(E, dtype=jnp.int32), NB)
    lo = jnp.maximum(off[ee], bb * B_R)
    hi = jnp.minimum(off[ee + 1], (bb + 1) * B_R)
    valid = lo < hi
    nvalid = jnp.sum(valid.astype(jnp.int32))
    rank = jnp.cumsum(valid.astype(jnp.int32)) - 1
    dest = jnp.where(valid, rank, T_ITEMS)
    bids = jnp.zeros((T_ITEMS,), jnp.int32).at[dest].set(bb, mode="drop")
    eids = jnp.zeros((T_ITEMS,), jnp.int32).at[dest].set(ee, mode="drop")
    glo = jnp.zeros((T_ITEMS,), jnp.int32).at[dest].set(lo, mode="drop")
    ghi = jnp.zeros((T_ITEMS,), jnp.int32).at[dest].set(hi, mode="drop")
    tpos = jnp.arange(T_ITEMS, dtype=jnp.int32)
    pad = tpos >= nvalid
    lb = jnp.take(bids, nvalid - 1)
    le = jnp.take(eids, nvalid - 1)
    lh = jnp.take(ghi, nvalid - 1)
    bids = jnp.where(pad, lb, bids)
    eids = jnp.where(pad, le, eids)
    glo = jnp.where(pad, lh, glo)
    ghi = jnp.where(pad, lh, ghi)
    finit = (jnp.concatenate([jnp.full((1,), -1, jnp.int32), bids[:-1]])
             != bids).astype(jnp.int32)
    return bids, eids, glo, ghi, finit


def _combine_body(y, posj, pw, out, pos_v, pw_v, rows_v, out_v, sem):
    c = lax.axis_index("c")
    s = lax.axis_index("s")
    wid = s * 2 + c
    base = wid * CH
    tok0 = wid * TOKW

    pltpu.sync_copy(pw.at[pl.ds(base, CH)], pw_v)
    iota = lax.iota(jnp.int32, 16)

    for half in range(2):
        pltpu.sync_copy(posj.at[pl.ds(base + half * 64, 64)], pos_v.at[half])
        pltpu.async_copy(y.at[pos_v.at[half]], rows_v, sem).wait()

        def token_body(i, _):
            a0 = half * 64 + 2 * i
            q = pw_v[pl.ds((a0 // 16) * 16, 16)]
            l0 = a0 % 16
            w0 = q.at[iota * 0 + l0].get(mode="promise_in_bounds")
            w1 = q.at[iota * 0 + l0 + 1].get(mode="promise_in_bounds")
            r0 = 2 * i
            for v in range(D_MODEL // 16):
                x0 = rows_v[r0, pl.ds(v * 16, 16)]
                x1 = rows_v[r0 + 1, pl.ds(v * 16, 16)]
                out_v[i, pl.ds(v * 16, 16)] = w0 * x0 + w1 * x1
            return 0

        lax.fori_loop(0, 32, token_body, 0)
        pltpu.sync_copy(out_v, out.at[pl.ds(tok0 + half * 32, 32)])


def _combine_call(y, posj, pwflat):
    mesh = plsc.VectorSubcoreMesh(core_axis_name="c", subcore_axis_name="s")
    return pl.kernel(
        _combine_body,
        out_type=jax.ShapeDtypeStruct((N_TOK, D_MODEL), jnp.float32),
        mesh=mesh,
        scratch_types=[
            pltpu.VMEM((2, 64), jnp.int32),
            pltpu.VMEM((CH,), jnp.float32),
            pltpu.VMEM((64, D_MODEL), jnp.float32),
            pltpu.VMEM((32, D_MODEL), jnp.float32),
            pltpu.SemaphoreType.DMA,
        ],
    )(y, posj, pwflat)


def kernel(x, W_router, W_gate, W_up, W_down):
    B, S, D = x.shape
    xf = x.reshape(S, D)
    idx2, p2, counts, aux, starts = _router_call(xf, W_router)

    xs, posj = _dispatch_call(idx2.reshape(N_ASSIGN), xf, starts)

    meta = _metadata(counts.reshape(E))
    y = _ffn_call(meta, xs, W_gate, W_up, W_down)

    out = _combine_call(y, posj, p2.reshape(N_ASSIGN))
    return out.reshape(B, S, D), aux.reshape(())
